# baseline (speedup 1.0000x reference)
import functools

import jax
import jax.numpy as jnp
from jax import lax
from jax.experimental import pallas as pl
from jax.experimental.pallas import tpu as pltpu
from jax.experimental.pallas import tpu_sc as plsc

_B = 1024
_D = 128
_SEQ = 50
_NC = 2
_NS = 16
_NW = _NC * _NS
_BPW = _B // _NW

_mesh = plsc.VectorSubcoreMesh(core_axis_name="c", subcore_axis_name="s")


@functools.partial(
    pl.kernel,
    mesh=_mesh,
    out_type=[
        jax.ShapeDtypeStruct((_B, _D), jnp.float32),
        jax.ShapeDtypeStruct((_B, _D), jnp.float32),
    ],
    scratch_types=[
        pltpu.VMEM((2 * _BPW,), jnp.int32),
        pltpu.VMEM((2 * _BPW, _D), jnp.float32),
        pltpu.SemaphoreType.DMA,
    ],
)
def _sc_double_gather(idx_hbm, table_hbm, apb_out, lsn_out,
                      idx_v, rows_v, sem_g):
    wid = lax.axis_index("s") * _NC + lax.axis_index("c")
    base = wid * _BPW
    pltpu.sync_copy(idx_hbm.at[pl.ds(wid * 2 * _BPW, 2 * _BPW)], idx_v)
    pltpu.async_copy(table_hbm.at[idx_v], rows_v, sem_g).wait()
    pltpu.sync_copy(rows_v.at[pl.ds(0, _BPW)], apb_out.at[pl.ds(base, _BPW)])
    pltpu.sync_copy(rows_v.at[pl.ds(_BPW, _BPW)], lsn_out.at[pl.ds(base, _BPW)])


def kernel(spkemb, alsn, tlsn, active_passive_bit, lsn_id, ape_table, lsn_table):
    table = jnp.concatenate([ape_table, lsn_table], axis=0)
    apb_i = active_passive_bit.astype(jnp.int32).reshape(_NW, 1, _BPW)
    lsn_i = (lsn_id.astype(jnp.int32) + 3).reshape(_NW, 1, _BPW)
    idx = jnp.concatenate([apb_i, lsn_i], axis=1).reshape(-1)
    c = lax.optimization_barrier(jnp.zeros((), jnp.int32))
    spk_d, alsn_d, idx = lax.optimization_barrier((spkemb, alsn, idx))
    apb, lsn_rows = _sc_double_gather(idx, table)
    spk_o = lax.dynamic_update_slice(spk_d, spk_d[:1], (c, c, c))
    alsn_o = lax.dynamic_update_slice(alsn_d, alsn_d[:1], (c, c, c))
    tlsn_d, apb = lax.optimization_barrier((tlsn, apb))
    tlsn_o = lax.dynamic_update_slice(tlsn_d, tlsn_d[:1], (c, c, c))
    return (spk_o, alsn_o, tlsn_o, apb, lsn_rows[:, None, :])


# speedup vs baseline: 1.0327x; 1.0327x over previous
import functools

import jax
import jax.numpy as jnp
from jax import lax
from jax.experimental import pallas as pl
from jax.experimental.pallas import tpu as pltpu
from jax.experimental.pallas import tpu_sc as plsc

_B = 1024
_D = 128
_SEQ = 50
_NC = 2
_NS = 16
_NW = _NC * _NS
_BPW = _B // _NW

_mesh = plsc.VectorSubcoreMesh(core_axis_name="c", subcore_axis_name="s")


@functools.partial(
    pl.kernel,
    mesh=_mesh,
    out_type=[
        jax.ShapeDtypeStruct((_B, _D), jnp.float32),
        jax.ShapeDtypeStruct((_B, _D), jnp.float32),
    ],
    scratch_types=[
        pltpu.VMEM((2 * _BPW,), jnp.int32),
        pltpu.VMEM((2 * _BPW, _D), jnp.float32),
        pltpu.SemaphoreType.DMA,
    ],
)
def _sc_double_gather(idx_hbm, table_hbm, apb_out, lsn_out,
                      idx_v, rows_v, sem_g):
    wid = lax.axis_index("s") * _NC + lax.axis_index("c")
    base = wid * _BPW
    pltpu.sync_copy(idx_hbm.at[pl.ds(wid * 2 * _BPW, 2 * _BPW)], idx_v)
    pltpu.async_copy(table_hbm.at[idx_v], rows_v, sem_g).wait()
    pltpu.sync_copy(rows_v.at[pl.ds(0, _BPW)], apb_out.at[pl.ds(base, _BPW)])
    pltpu.sync_copy(rows_v.at[pl.ds(_BPW, _BPW)], lsn_out.at[pl.ds(base, _BPW)])


def kernel(spkemb, alsn, tlsn, active_passive_bit, lsn_id, ape_table, lsn_table):
    table = jnp.concatenate([ape_table, lsn_table], axis=0)
    apb_i = active_passive_bit.astype(jnp.int32).reshape(_NW, 1, _BPW)
    lsn_i = (lsn_id.astype(jnp.int32) + 3).reshape(_NW, 1, _BPW)
    idx = jnp.concatenate([apb_i, lsn_i], axis=1).reshape(-1)
    z = lax.optimization_barrier(jnp.zeros((), jnp.float32))
    spk_o = spkemb + z
    alsn_o = alsn + z
    apb, lsn_rows = _sc_double_gather(idx, table)
    tlsn_d, apb = lax.optimization_barrier((tlsn, apb))
    tlsn_o = tlsn_d + z
    return (spk_o, alsn_o, tlsn_o, apb, lsn_rows[:, None, :])


# speedup vs baseline: 1.0439x; 1.0109x over previous
import functools

import jax
import jax.numpy as jnp
from jax import lax
from jax.experimental import pallas as pl
from jax.experimental.pallas import tpu as pltpu
from jax.experimental.pallas import tpu_sc as plsc

_B = 1024
_D = 128
_SEQ = 50
_NC = 2
_NS = 16
_NW = _NC * _NS
_BPW = _B // _NW

_mesh = plsc.VectorSubcoreMesh(core_axis_name="c", subcore_axis_name="s")


@functools.partial(
    pl.kernel,
    mesh=_mesh,
    out_type=[
        jax.ShapeDtypeStruct((_B, _D), jnp.float32),
        jax.ShapeDtypeStruct((_B, _D), jnp.float32),
    ],
    scratch_types=[
        pltpu.VMEM((2 * _BPW,), jnp.int32),
        pltpu.VMEM((2 * _BPW, _D), jnp.float32),
        pltpu.SemaphoreType.DMA,
    ],
)
def _sc_double_gather(idx_hbm, table_hbm, apb_out, lsn_out,
                      idx_v, rows_v, sem_g):
    wid = lax.axis_index("s") * _NC + lax.axis_index("c")
    base = wid * _BPW
    pltpu.sync_copy(idx_hbm.at[pl.ds(wid * 2 * _BPW, 2 * _BPW)], idx_v)
    pltpu.async_copy(table_hbm.at[idx_v], rows_v, sem_g).wait()
    pltpu.sync_copy(rows_v.at[pl.ds(0, _BPW)], apb_out.at[pl.ds(base, _BPW)])
    pltpu.sync_copy(rows_v.at[pl.ds(_BPW, _BPW)], lsn_out.at[pl.ds(base, _BPW)])


def kernel(spkemb, alsn, tlsn, active_passive_bit, lsn_id, ape_table, lsn_table):
    table = jnp.concatenate([ape_table, lsn_table], axis=0)
    apb_i = active_passive_bit.astype(jnp.int32).reshape(_NW, 1, _BPW)
    lsn_i = (lsn_id.astype(jnp.int32) + 3).reshape(_NW, 1, _BPW)
    idx = jnp.concatenate([apb_i, lsn_i], axis=1).reshape(-1)
    z = lax.optimization_barrier(jnp.zeros((), jnp.float32))
    spk_o = spkemb + z
    alsn_o = alsn + z
    tlsn_o = tlsn + z
    idx, _ = lax.optimization_barrier((idx, spk_o[0, 0, 0]))
    apb, lsn_rows = _sc_double_gather(idx, table)
    return (spk_o, alsn_o, tlsn_o, apb, lsn_rows[:, None, :])
